# baseline (speedup 1.0000x reference)
import jax
import jax.numpy as jnp
from jax import lax
from jax.experimental import pallas as pl
from jax.experimental.pallas import tpu as pltpu
from jax.experimental.pallas import tpu_sc as plsc

B = 16
N = 4096
L = 16
CHUNKS = N // L
UNROLL = 8


def _sc_body(b_hbm, tid_hbm, rows_hbm, b_v, rows_v, tid_v):
    row = lax.axis_index("s")
    pltpu.sync_copy(b_hbm.at[row], b_v)
    pltpu.sync_copy(tid_hbm, tid_v)
    tid = tid_v[...]
    zeros_i = jnp.zeros((L,), jnp.int32)
    ones_i = jnp.ones((L,), jnp.int32)
    iota = lax.iota(jnp.int32, L)

    def body(g, n_vec):
        base = g * (UNROLL * L)
        offs = [base + k * L for k in range(UNROLL)]
        vs = [b_v[pl.ds(o, L)] for o in offs]
        ms = [v == tid for v in vs]
        cums = [plsc.cumsum(jnp.where(m, ones_i, zeros_i)) for m in ms]
        pcs = [plsc.all_reduce_population_count(m) for m in ms]
        n_k = n_vec
        for k in range(UNROLL):
            rows_v[pl.ds(offs[k], L)] = zeros_i
            pos = n_k + cums[k] - 1
            plsc.store_scatter(rows_v, [pos], iota + offs[k], mask=ms[k])
            n_k = n_k + pcs[k]
        return n_k

    lax.fori_loop(0, CHUNKS // UNROLL, body, zeros_i)
    pltpu.sync_copy(rows_v, rows_hbm.at[row])


_sc_call = pl.kernel(
    _sc_body,
    out_type=jax.ShapeDtypeStruct((B, N), jnp.int32),
    mesh=plsc.VectorSubcoreMesh(
        core_axis_name="c", subcore_axis_name="s", num_cores=1),
    scratch_types=[
        pltpu.VMEM((N,), jnp.float32),
        pltpu.VMEM((N,), jnp.int32),
        pltpu.VMEM((L,), jnp.float32),
    ],
    compiler_params=pltpu.CompilerParams(
        needs_layout_passes=False, use_tc_tiling_on_sc=False),
)


def _tc_diff_body(b_ref, tid_ref, o_ref):
    o_ref[...] = jnp.where(
        b_ref[...] == tid_ref[0, 0], 1.0, 0.0)[..., None, None]


def _tc_diff(b, tid_arr):
    return pl.pallas_call(
        _tc_diff_body,
        out_shape=jax.ShapeDtypeStruct((B, N, 1, 1), jnp.float32),
        in_specs=[
            pl.BlockSpec((B, N), lambda: (0, 0)),
            pl.BlockSpec(memory_space=pltpu.SMEM),
        ],
        out_specs=pl.BlockSpec((B, N, 1, 1), lambda: (0, 0, 0, 0)),
    )(b, tid_arr)


def kernel(block_id, target_id):
    b = jnp.squeeze(block_id, -1)
    tidf = jnp.asarray(target_id, jnp.float32) + 1.0
    tid_vec = jnp.broadcast_to(tidf, (L,))
    rows = _sc_call(b, tid_vec)
    diff = _tc_diff(b, tidf.reshape(1, 1))
    return diff, rows


# speedup vs baseline: 5.3567x; 5.3567x over previous
import jax
import jax.numpy as jnp
from jax import lax
from jax.experimental import pallas as pl
from jax.experimental.pallas import tpu as pltpu
from jax.experimental.pallas import tpu_sc as plsc

B = 16
N = 4096
L = 16
CHUNKS = N // L
UNROLL = 8


def _sc_body(b_hbm, tid_hbm, rows_hbm, b_v, rows_v, tid_v):
    row = lax.axis_index("s")
    pltpu.sync_copy(b_hbm.at[row], b_v)
    pltpu.sync_copy(tid_hbm, tid_v)
    tid = tid_v[...]
    zeros_i = jnp.zeros((L,), jnp.int32)
    ones_i = jnp.ones((L,), jnp.int32)
    iota = lax.iota(jnp.int32, L)

    def body(g, n_vec):
        base = g * (UNROLL * L)
        offs = [base + k * L for k in range(UNROLL)]
        vs = [b_v[pl.ds(o, L)] for o in offs]
        ms = [v == tid for v in vs]
        cums = [plsc.cumsum(jnp.where(m, ones_i, zeros_i)) for m in ms]
        pcs = [plsc.all_reduce_population_count(m) for m in ms]
        n_k = n_vec
        for k in range(UNROLL):
            rows_v[pl.ds(offs[k], L)] = zeros_i
            pos = n_k + cums[k] - 1
            plsc.store_scatter(rows_v, [pos], iota + offs[k], mask=ms[k])
            n_k = n_k + pcs[k]
        return n_k

    lax.fori_loop(0, CHUNKS // UNROLL, body, zeros_i)
    pltpu.sync_copy(rows_v, rows_hbm.at[row])


_sc_call = pl.kernel(
    _sc_body,
    out_type=jax.ShapeDtypeStruct((B, N), jnp.int32),
    mesh=plsc.VectorSubcoreMesh(
        core_axis_name="c", subcore_axis_name="s", num_cores=1),
    scratch_types=[
        pltpu.VMEM((N,), jnp.float32),
        pltpu.VMEM((N,), jnp.int32),
        pltpu.VMEM((L,), jnp.float32),
    ],
    compiler_params=pltpu.CompilerParams(
        needs_layout_passes=False, use_tc_tiling_on_sc=False),
)


def _tc_diff_body(b_ref, tid_ref, o_ref):
    o_ref[...] = jnp.where(b_ref[...] == tid_ref[0, 0], 1.0, 0.0)


def _tc_diff(b, tid_arr):
    return pl.pallas_call(
        _tc_diff_body,
        out_shape=jax.ShapeDtypeStruct((B, N), jnp.float32),
        in_specs=[
            pl.BlockSpec((B, N), lambda: (0, 0)),
            pl.BlockSpec(memory_space=pltpu.SMEM),
        ],
        out_specs=pl.BlockSpec((B, N), lambda: (0, 0)),
    )(b, tid_arr)


def kernel(block_id, target_id):
    b = jnp.squeeze(block_id, -1)
    tidf = jnp.asarray(target_id, jnp.float32) + 1.0
    tid_vec = jnp.broadcast_to(tidf, (L,))
    rows = _sc_call(b, tid_vec)
    diff = _tc_diff(b, tidf.reshape(1, 1))
    return diff.reshape(B, N, 1, 1), rows
